# Initial kernel scaffold; baseline (speedup 1.0000x reference)
import jax
import jax.numpy as jnp
from jax.experimental import pallas as pl


def kernel(attn, W):
    raise NotImplementedError("write your pallas kernel here")


# speedup vs baseline: 35.6681x; 35.6681x over previous
import jax
import jax.numpy as jnp
from jax.experimental import pallas as pl

_N_HEADS = 16
_SPAN = (_N_HEADS - 1) // 2
_N_VALS = 2 * _SPAN + 1


def _bias_kernel(w_ref, o_ref, *, block_r, l):
    rb = pl.program_id(1)
    i0 = rb * block_r
    rows = jax.lax.broadcasted_iota(jnp.int32, (block_r, l), 0) + i0
    cols = jax.lax.broadcasted_iota(jnp.int32, (block_r, l), 1)
    d = cols - rows
    rp = jnp.clip(d, -_SPAN, _SPAN) + _SPAN
    acc = jnp.full((block_r, l), w_ref[0, 0, 0], dtype=jnp.float32)
    for k in range(1, _N_VALS):
        acc = jnp.where(rp == k, w_ref[0, 0, k], acc)
    o_ref[0, 0, :, :] = acc


def kernel(attn, W):
    n_heads = attn.shape[0]
    l = attn.shape[2]
    block_r = min(256, l)
    n_rb = l // block_r
    wt = W.T.reshape(n_heads, 1, n_heads).astype(jnp.float32)
    out = pl.pallas_call(
        lambda w_ref, o_ref: _bias_kernel(w_ref, o_ref, block_r=block_r, l=l),
        grid=(n_heads, n_rb),
        in_specs=[pl.BlockSpec((1, 1, n_heads), lambda h, rb: (h, 0, 0))],
        out_specs=pl.BlockSpec((1, 1, block_r, l), lambda h, rb: (h, 0, rb, 0)),
        out_shape=jax.ShapeDtypeStruct((n_heads, 1, l, l), jnp.float32),
    )(wt)
    return out
